# baseline (speedup 1.0000x reference)
import functools

import jax
import jax.numpy as jnp
from jax import lax
from jax.experimental import pallas as pl
from jax.experimental.pallas import tpu as pltpu
from jax.experimental.pallas import tpu_sc as plsc

HIDDEN = 128
CHUNK = 128
SETCHUNKS = 1
SETROWS = SETCHUNKS * CHUNK
SETS = 5
HBM_SETS = 1
NC = 2
NS = 16
NW = NC * NS


@functools.partial(jax.jit, static_argnums=(0, 1))
def _lookup(n_tokens, chunks_per_w, idx, table):
    per_w = chunks_per_w * CHUNK
    groups = chunks_per_w // SETCHUNKS
    rounds = groups // SETS
    vocab = table.shape[0]
    stage_rows = vocab // NS
    mesh = plsc.VectorSubcoreMesh(core_axis_name="c", subcore_axis_name="s")

    @functools.partial(
        pl.kernel,
        mesh=mesh,
        out_type=jax.ShapeDtypeStruct((n_tokens, HIDDEN), jnp.float32),
        scratch_types=[
            pltpu.VMEM((chunks_per_w, CHUNK), jnp.int32),
            pltpu.VMEM((SETS, SETROWS, HIDDEN), jnp.float32),
            pltpu.VMEM_SHARED((vocab, HIDDEN), jnp.float32),
        ]
        + [pltpu.SemaphoreType.DMA] * (2 * SETS),
    )
    def k(idx_hbm, table_hbm, out_hbm, idx_v, rows_v, table_sh, *sems):
        gsem = sems[:SETS]
        wsem = sems[SETS:]
        sid = lax.axis_index("s")
        wid = sid * NC + lax.axis_index("c")
        base = wid * per_w
        pltpu.sync_copy(
            table_hbm.at[pl.ds(sid * stage_rows, stage_rows)],
            table_sh.at[pl.ds(sid * stage_rows, stage_rows)],
        )
        pltpu.sync_copy(idx_hbm.at[wid], idx_v)
        plsc.subcore_barrier()

        def gather_descs(sp, g):
            src_table = table_hbm if sp >= SETS - HBM_SETS else table_sh
            return [
                pltpu.make_async_copy(
                    src_table.at[idx_v.at[g * SETCHUNKS + b]],
                    rows_v.at[sp, pl.ds(b * CHUNK, CHUNK)],
                    gsem[sp],
                )
                for b in range(SETCHUNKS)
            ]

        def write_desc(sp, g):
            return pltpu.make_async_copy(
                rows_v.at[sp],
                out_hbm.at[pl.ds(base + g * SETROWS, SETROWS)],
                wsem[sp],
            )

        def issue_gathers(sp, g):
            for d in gather_descs(sp, g):
                d.start()

        for sp in range(SETS):
            issue_gathers(sp, sp)

        def body(t, carry):
            for sp in range(SETS):
                for d in gather_descs(sp, SETS * t + sp):
                    d.wait()
                write_desc(sp, SETS * t + sp).start()
            for sp in range(SETS):
                write_desc(sp, SETS * t + sp).wait()
                issue_gathers(sp, SETS * (t + 1) + sp)
            return carry

        lax.fori_loop(0, rounds - 1, body, 0)

        t = rounds - 1
        wds = []
        for sp in range(SETS):
            for d in gather_descs(sp, SETS * t + sp):
                d.wait()
            wd = write_desc(sp, SETS * t + sp)
            wd.start()
            wds.append(wd)
        for wd in wds:
            wd.wait()

    return k(idx, table)


def kernel(batch, doc_len, embed_weight):
    del doc_len
    bsz, seq = batch.shape
    n_tokens = bsz * seq
    chunks_per_w = n_tokens // (NW * CHUNK)
    idx = batch.reshape(NW, chunks_per_w, CHUNK).astype(jnp.int32)
    vocab = embed_weight.shape[0]
    vpad = -(-vocab // 128) * 128
    if vpad != vocab:
        embed_weight = jnp.pad(embed_weight, ((0, vpad - vocab), (0, 0)))
    out = _lookup(n_tokens, chunks_per_w, idx, embed_weight)
    return out.reshape(bsz, seq, HIDDEN)


# speedup vs baseline: 1.1925x; 1.1925x over previous
import functools

import jax
import jax.numpy as jnp
from jax import lax
from jax.experimental import pallas as pl
from jax.experimental.pallas import tpu as pltpu
from jax.experimental.pallas import tpu_sc as plsc

HIDDEN = 128
CHUNK = 80
SETCHUNKS = 2
SETROWS = SETCHUNKS * CHUNK
SETS = 4
NC = 2
NS = 16
NW = NC * NS


@functools.partial(jax.jit, static_argnums=(0, 1))
def _lookup(n_tokens, chunks_per_w, idx, table):
    per_w = chunks_per_w * CHUNK
    groups = chunks_per_w // SETCHUNKS
    rounds = groups // SETS
    vocab = table.shape[0]
    n_full = vocab // 128
    rem = vocab % 128
    mesh = plsc.VectorSubcoreMesh(core_axis_name="c", subcore_axis_name="s")

    @functools.partial(
        pl.kernel,
        mesh=mesh,
        out_type=jax.ShapeDtypeStruct((n_tokens, HIDDEN), jnp.float32),
        scratch_types=[
            pltpu.VMEM((chunks_per_w, CHUNK), jnp.int32),
            pltpu.VMEM((SETS, SETROWS, HIDDEN), jnp.float32),
            pltpu.VMEM_SHARED((vocab, HIDDEN), jnp.float32),
        ]
        + [pltpu.SemaphoreType.DMA] * (2 * SETS),
    )
    def k(idx_hbm, table_hbm, out_hbm, idx_v, rows_v, table_sh, *sems):
        gsem = sems[:SETS]
        wsem = sems[SETS:]
        sid = lax.axis_index("s")
        wid = sid * NC + lax.axis_index("c")
        base = wid * per_w
        @pl.when(sid < n_full)
        def _():
            pltpu.sync_copy(
                table_hbm.at[pl.ds(sid * 128, 128)],
                table_sh.at[pl.ds(sid * 128, 128)],
            )

        if rem:
            @pl.when(sid == n_full)
            def _():
                pltpu.sync_copy(
                    table_hbm.at[pl.ds(n_full * 128, rem)],
                    table_sh.at[pl.ds(n_full * 128, rem)],
                )
        pltpu.sync_copy(idx_hbm.at[wid], idx_v)
        plsc.subcore_barrier()

        def gather_descs(sp, g):
            return [
                pltpu.make_async_copy(
                    table_sh.at[idx_v.at[g * SETCHUNKS + b]],
                    rows_v.at[sp, pl.ds(b * CHUNK, CHUNK)],
                    gsem[sp],
                )
                for b in range(SETCHUNKS)
            ]

        def write_desc(sp, g):
            return pltpu.make_async_copy(
                rows_v.at[sp],
                out_hbm.at[pl.ds(base + g * SETROWS, SETROWS)],
                wsem[sp],
            )

        def issue_gathers(sp, g):
            for d in gather_descs(sp, g):
                d.start()

        for sp in range(SETS):
            issue_gathers(sp, sp)

        def body(t, carry):
            for sp in range(SETS):
                for d in gather_descs(sp, SETS * t + sp):
                    d.wait()
                write_desc(sp, SETS * t + sp).start()
            for sp in range(SETS):
                write_desc(sp, SETS * t + sp).wait()
                issue_gathers(sp, SETS * (t + 1) + sp)
            return carry

        lax.fori_loop(0, rounds - 1, body, 0)

        t = rounds - 1
        wds = []
        for sp in range(SETS):
            for d in gather_descs(sp, SETS * t + sp):
                d.wait()
            wd = write_desc(sp, SETS * t + sp)
            wd.start()
            wds.append(wd)
        for wd in wds:
            wd.wait()

    return k(idx, table)


def kernel(batch, doc_len, embed_weight):
    del doc_len
    bsz, seq = batch.shape
    n_tokens = bsz * seq
    chunks_per_w = n_tokens // (NW * CHUNK)
    idx = batch.reshape(NW, chunks_per_w, CHUNK).astype(jnp.int32)
    out = _lookup(n_tokens, chunks_per_w, idx, embed_weight)
    return out.reshape(bsz, seq, HIDDEN)
